# Initial kernel scaffold; baseline (speedup 1.0000x reference)
import functools

import jax
import jax.numpy as jnp
from jax import lax
from jax.experimental import pallas as pl
from jax.experimental.pallas import tpu as pltpu
from jax.experimental.pallas import tpu_sc as plsc

N = 10000
E = 320000
D = 128
H = 64
G = 64

NPAD = 10240
K = 128
NCHUNK = E // K
NW = 32
TPW = (NCHUNK + NW - 1) // NW
RPT = NPAD // NW


def _segsum_body(y_hbm, src_hbm, dst_hbm, zeros_hbm, out_hbm,
                 acc_sh, src_v, dst_v, rows_v, zbuf_v, sem):
    c = lax.axis_index("c")
    s = lax.axis_index("s")
    wid = s * 2 + c
    pltpu.sync_copy(zeros_hbm.at[pl.ds(s * RPT, RPT)], zbuf_v)
    pltpu.sync_copy(zbuf_v, acc_sh.at[pl.ds(s * RPT, RPT)])
    plsc.subcore_barrier()

    @pl.loop(0, TPW)
    def _edge_loop(t):
        chunk = wid + t * NW

        @pl.when(chunk < NCHUNK)
        def _():
            base = chunk * K
            pltpu.sync_copy(src_hbm.at[pl.ds(base, K)], src_v)
            pltpu.sync_copy(dst_hbm.at[pl.ds(base, K)], dst_v)
            pltpu.async_copy(y_hbm.at[src_v], rows_v, sem).wait()
            pltpu.sync_copy(rows_v, acc_sh.at[dst_v], add=True)

    plsc.subcore_barrier()
    pltpu.sync_copy(acc_sh.at[pl.ds(s * RPT, RPT)], zbuf_v)
    pltpu.sync_copy(zbuf_v, out_hbm.at[c, pl.ds(s * RPT, RPT)])


_segsum = pl.kernel(
    _segsum_body,
    out_type=jax.ShapeDtypeStruct((2, NPAD, H), jnp.float32),
    mesh=plsc.VectorSubcoreMesh(core_axis_name="c", subcore_axis_name="s"),
    scratch_types=[
        pltpu.VMEM_SHARED((NPAD, H), jnp.float32),
        pltpu.VMEM((K,), jnp.int32),
        pltpu.VMEM((K,), jnp.int32),
        pltpu.VMEM((K, H), jnp.float32),
        pltpu.VMEM((RPT, H), jnp.float32),
        pltpu.SemaphoreType.DMA,
    ],
)


def _proj_body(x_ref, w_ref, o_ref):
    o_ref[...] = jnp.dot(x_ref[...], w_ref[...],
                         preferred_element_type=jnp.float32)


def _proj(x, w):
    return pl.pallas_call(
        _proj_body,
        out_shape=jax.ShapeDtypeStruct((x.shape[0], w.shape[1]), jnp.float32),
    )(x, w)


def _layer_body(y_ref, agg_ref, b1_ref, w2_ref, b2_ref, g_ref, be_ref,
                wn_ref, o_ref):
    z = y_ref[...] + agg_ref[0, :N, :] + agg_ref[1, :N, :] + b1_ref[...]
    u = jnp.maximum(z, 0.0)
    v = jnp.maximum(
        jnp.dot(u, w2_ref[...], preferred_element_type=jnp.float32)
        + b2_ref[...], 0.0)
    mean = jnp.mean(v, axis=0, keepdims=True)
    var = jnp.mean((v - mean) ** 2, axis=0, keepdims=True)
    hn = (v - mean) / jnp.sqrt(var + 1e-5) * g_ref[...] + be_ref[...]
    h = jnp.maximum(hn, 0.0)
    o_ref[...] = jnp.dot(h, wn_ref[...], preferred_element_type=jnp.float32)


def _layer(y, agg, b1, w2, b2, gamma, beta, wnext):
    return pl.pallas_call(
        _layer_body,
        out_shape=jax.ShapeDtypeStruct((N, wnext.shape[1]), jnp.float32),
    )(y, agg, b1.reshape(1, H), w2, b2.reshape(1, H),
      gamma.reshape(1, H), beta.reshape(1, H), wnext)


def _final_body(y_ref, agg_ref, b1_ref, w2_ref, b2_ref, g_ref, be_ref,
                batch_ref, fcw_ref, fcb_ref, o_ref):
    z = y_ref[...] + agg_ref[0, :N, :] + agg_ref[1, :N, :] + b1_ref[...]
    u = jnp.maximum(z, 0.0)
    v = jnp.maximum(
        jnp.dot(u, w2_ref[...], preferred_element_type=jnp.float32)
        + b2_ref[...], 0.0)
    mean = jnp.mean(v, axis=0, keepdims=True)
    var = jnp.mean((v - mean) ** 2, axis=0, keepdims=True)
    hn = (v - mean) / jnp.sqrt(var + 1e-5) * g_ref[...] + be_ref[...]
    h = jnp.maximum(hn, 0.0)
    seg = lax.broadcasted_iota(jnp.int32, (G, N), 0)
    mask = (seg == batch_ref[...]).astype(jnp.float32)
    pooled = jnp.dot(mask, h, preferred_element_type=jnp.float32)
    o_ref[...] = jnp.dot(pooled, fcw_ref[...],
                         preferred_element_type=jnp.float32) + fcb_ref[...]


def _final(y, agg, b1, w2, b2, gamma, beta, batch, fc_W, fc_b):
    return pl.pallas_call(
        _final_body,
        out_shape=jax.ShapeDtypeStruct((G, 1), jnp.float32),
    )(y, agg, b1.reshape(1, H), w2, b2.reshape(1, H),
      gamma.reshape(1, H), beta.reshape(1, H),
      batch.reshape(1, N).astype(jnp.int32), fc_W, fc_b.reshape(1, 1))


def kernel(x, edge_index, batch,
           c1_W1, c1_b1, c1_W2, c1_b2, c1_gamma, c1_beta,
           c2_W1, c2_b1, c2_W2, c2_b2, c2_gamma, c2_beta,
           c3_W1, c3_b1, c3_W2, c3_b2, c3_gamma, c3_beta,
           fc_W, fc_b):
    src = edge_index[0].astype(jnp.int32)
    dst = edge_index[1].astype(jnp.int32)
    zeros = jnp.zeros((NPAD, H), jnp.float32)

    y1 = _proj(x, c1_W1)
    a1 = _segsum(y1, src, dst, zeros)
    y2 = _layer(y1, a1, c1_b1, c1_W2, c1_b2, c1_gamma, c1_beta, c2_W1)
    a2 = _segsum(y2, src, dst, zeros)
    y3 = _layer(y2, a2, c2_b1, c2_W2, c2_b2, c2_gamma, c2_beta, c3_W1)
    a3 = _segsum(y3, src, dst, zeros)
    out = _final(y3, a3, c3_b1, c3_W2, c3_b2, c3_gamma, c3_beta,
                 batch, fc_W, fc_b)
    return out


# speedup vs baseline: 5.9097x; 5.9097x over previous
import functools

import jax
import jax.numpy as jnp
from jax import lax
from jax.experimental import pallas as pl
from jax.experimental.pallas import tpu as pltpu
from jax.experimental.pallas import tpu_sc as plsc

N = 10000
E = 320000
D = 128
H = 64
G = 64

NPAD = 10240
K = 128
NW = 32
TPW = (E // K + NW - 1) // NW
EPAD = NW * TPW * K
RPT = NPAD // 16


def _segsum_body(y_hbm, src_hbm, dst_hbm, zeros_hbm, out_hbm,
                 acc_sh, src_v, dst_v, rows_v, zbuf_v, sem):
    c = lax.axis_index("c")
    s = lax.axis_index("s")
    wid = s * 2 + c
    pltpu.sync_copy(zeros_hbm.at[pl.ds(s * RPT, RPT)], zbuf_v)
    pltpu.sync_copy(zbuf_v, acc_sh.at[pl.ds(s * RPT, RPT)])
    pltpu.sync_copy(src_hbm.at[pl.ds(wid * TPW, TPW)], src_v)
    pltpu.sync_copy(dst_hbm.at[pl.ds(wid * TPW, TPW)], dst_v)
    plsc.subcore_barrier()

    @pl.loop(0, TPW)
    def _edge_loop(t):
        pltpu.async_copy(y_hbm.at[src_v.at[t]], rows_v, sem).wait()
        pltpu.sync_copy(rows_v, acc_sh.at[dst_v.at[t]], add=True)

    plsc.subcore_barrier()
    pltpu.sync_copy(acc_sh.at[pl.ds(s * RPT, RPT)], zbuf_v)
    pltpu.sync_copy(zbuf_v, out_hbm.at[c, pl.ds(s * RPT, RPT)])


@functools.cache
def _get_segsum():
    return pl.kernel(
        _segsum_body,
        out_type=jax.ShapeDtypeStruct((2, NPAD, H), jnp.float32),
        mesh=plsc.VectorSubcoreMesh(core_axis_name="c", subcore_axis_name="s"),
        compiler_params=pltpu.CompilerParams(use_tc_tiling_on_sc=False),
        scratch_types=[
            pltpu.VMEM_SHARED((NPAD, H), jnp.float32),
            pltpu.VMEM((TPW, K), jnp.int32),
            pltpu.VMEM((TPW, K), jnp.int32),
            pltpu.VMEM((K, H), jnp.float32),
            pltpu.VMEM((RPT, H), jnp.float32),
            pltpu.SemaphoreType.DMA,
        ],
    )


def _segsum(y, src2, dst2, zeros):
    return _get_segsum()(y, src2, dst2, zeros)


def _proj_body(x_ref, w_ref, o_ref):
    o_ref[...] = jnp.dot(x_ref[...], w_ref[...],
                         preferred_element_type=jnp.float32,
                 precision=jax.lax.Precision.HIGHEST)


def _proj(x, w):
    return pl.pallas_call(
        _proj_body,
        out_shape=jax.ShapeDtypeStruct((x.shape[0], w.shape[1]), jnp.float32),
    )(x, w)


def _layer_body(y_ref, agg_ref, b1_ref, w2_ref, b2_ref, g_ref, be_ref,
                wn_ref, o_ref):
    z = y_ref[...] + agg_ref[0, :N, :] + agg_ref[1, :N, :] + b1_ref[...]
    u = jnp.maximum(z, 0.0)
    v = jnp.maximum(
        jnp.dot(u, w2_ref[...], preferred_element_type=jnp.float32,
                 precision=jax.lax.Precision.HIGHEST)
        + b2_ref[...], 0.0)
    mean = jnp.mean(v, axis=0, keepdims=True)
    var = jnp.mean((v - mean) ** 2, axis=0, keepdims=True)
    hn = (v - mean) / jnp.sqrt(var + 1e-5) * g_ref[...] + be_ref[...]
    h = jnp.maximum(hn, 0.0)
    o_ref[...] = jnp.dot(h, wn_ref[...], preferred_element_type=jnp.float32,
                 precision=jax.lax.Precision.HIGHEST)


def _layer(y, agg, b1, w2, b2, gamma, beta, wnext):
    return pl.pallas_call(
        _layer_body,
        out_shape=jax.ShapeDtypeStruct((N, wnext.shape[1]), jnp.float32),
    )(y, agg, b1.reshape(1, H), w2, b2.reshape(1, H),
      gamma.reshape(1, H), beta.reshape(1, H), wnext)


def _final_body(y_ref, agg_ref, b1_ref, w2_ref, b2_ref, g_ref, be_ref,
                batch_ref, fcw_ref, fcb_ref, o_ref):
    z = y_ref[...] + agg_ref[0, :N, :] + agg_ref[1, :N, :] + b1_ref[...]
    u = jnp.maximum(z, 0.0)
    v = jnp.maximum(
        jnp.dot(u, w2_ref[...], preferred_element_type=jnp.float32,
                 precision=jax.lax.Precision.HIGHEST)
        + b2_ref[...], 0.0)
    mean = jnp.mean(v, axis=0, keepdims=True)
    var = jnp.mean((v - mean) ** 2, axis=0, keepdims=True)
    hn = (v - mean) / jnp.sqrt(var + 1e-5) * g_ref[...] + be_ref[...]
    h = jnp.maximum(hn, 0.0)
    seg = lax.broadcasted_iota(jnp.int32, (G, N), 0)
    mask = (seg == batch_ref[...]).astype(jnp.float32)
    pooled = jnp.dot(mask, h, preferred_element_type=jnp.float32,
                 precision=jax.lax.Precision.HIGHEST)
    o_ref[...] = jnp.dot(pooled, fcw_ref[...],
                         preferred_element_type=jnp.float32,
                 precision=jax.lax.Precision.HIGHEST) + fcb_ref[...]


def _final(y, agg, b1, w2, b2, gamma, beta, batch, fc_W, fc_b):
    return pl.pallas_call(
        _final_body,
        out_shape=jax.ShapeDtypeStruct((G, 1), jnp.float32),
    )(y, agg, b1.reshape(1, H), w2, b2.reshape(1, H),
      gamma.reshape(1, H), beta.reshape(1, H),
      batch.reshape(1, N).astype(jnp.int32), fc_W, fc_b.reshape(1, 1))


def kernel(x, edge_index, batch,
           c1_W1, c1_b1, c1_W2, c1_b2, c1_gamma, c1_beta,
           c2_W1, c2_b1, c2_W2, c2_b2, c2_gamma, c2_beta,
           c3_W1, c3_b1, c3_W2, c3_b2, c3_gamma, c3_beta,
           fc_W, fc_b):
    src = edge_index[0].astype(jnp.int32)
    dst = edge_index[1].astype(jnp.int32)
    src = jnp.pad(src, (0, EPAD - E)).reshape(NW * TPW, K)
    dst = jnp.pad(dst, (0, EPAD - E),
                  constant_values=NPAD - 1).reshape(NW * TPW, K)
    zeros = jnp.zeros((NPAD, H), jnp.float32)

    y1 = _proj(x, c1_W1)
    a1 = _segsum(y1, src, dst, zeros)
    y2 = _layer(y1, a1, c1_b1, c1_W2, c1_b2, c1_gamma, c1_beta, c2_W1)
    a2 = _segsum(y2, src, dst, zeros)
    y3 = _layer(y2, a2, c2_b1, c2_W2, c2_b2, c2_gamma, c2_beta, c3_W1)
    a3 = _segsum(y3, src, dst, zeros)
    out = _final(y3, a3, c3_b1, c3_W2, c3_b2, c3_gamma, c3_beta,
                 batch, fc_W, fc_b)
    return out
